# baseline (speedup 1.0000x reference)
import functools

import jax
import jax.numpy as jnp
from jax import lax
from jax.experimental import pallas as pl
from jax.experimental.pallas import tpu as pltpu
from jax.experimental.pallas import tpu_sc as plsc

N = 16384
K = 16
TOTAL = N * K
NC = 2
NS = 16
NW = NC * NS
NH = N
AW = NH // NW
GROUPS = AW // 16
WIN = 8192
ROWS = 52
BN = 1024


def _sc_gather_body(tab_hbm, nl_hbm, beg_hbm, con_hbm, out_hbm,
                    tab, nlw, beg, con, ob, tsem, half):
    wid = lax.axis_index("s") * NC + lax.axis_index("c")
    a0l = wid * AW
    a0 = a0l + half * NH
    tab_cp = pltpu.async_copy(tab_hbm, tab, tsem)
    pltpu.sync_copy(beg_hbm.at[pl.ds(a0, AW)], beg)
    pltpu.sync_copy(con_hbm.at[pl.ds(a0, AW)], con)
    b0 = beg[pl.ds(0, 16)][0]
    off = jnp.minimum((b0 // 8) * 8, TOTAL - WIN)
    off = pl.multiple_of(off, 8)
    pltpu.sync_copy(nl_hbm.at[pl.ds(off, WIN)], nlw)
    tab_cp.wait()
    lane = lax.iota(jnp.int32, 16)

    @plsc.parallel_loop(0, GROUPS, unroll=16)
    def group(g):
        base = g * 16
        aidx = (a0 + base + lane) * 3
        ob[3 * K + 0, pl.ds(base, 16)] = plsc.load_gather(tab, [aidx])
        ob[3 * K + 1, pl.ds(base, 16)] = plsc.load_gather(tab, [aidx + 1])
        ob[3 * K + 2, pl.ds(base, 16)] = plsc.load_gather(tab, [aidx + 2])
        ob[3 * K + 3, pl.ds(base, 16)] = con[pl.ds(base, 16)].astype(jnp.float32)
        bvec = jnp.clip(beg[pl.ds(base, 16)] - off, 0, WIN - K)
        for s in range(K):
            nlv = plsc.load_gather(nlw, [bvec + s]) * 3
            ob[s, pl.ds(base, 16)] = plsc.load_gather(tab, [nlv])
            ob[K + s, pl.ds(base, 16)] = plsc.load_gather(tab, [nlv + 1])
            ob[2 * K + s, pl.ds(base, 16)] = plsc.load_gather(tab, [nlv + 2])

    pltpu.sync_copy(ob, out_hbm.at[:, pl.ds(a0l, AW)])


@functools.lru_cache(maxsize=2)
def _sc_gather(half):
    return pl.kernel(
        functools.partial(_sc_gather_body, half=half),
        out_type=jax.ShapeDtypeStruct((ROWS, NH), jnp.float32),
        mesh=plsc.VectorSubcoreMesh(core_axis_name="c", subcore_axis_name="s",
                                    num_cores=NC, num_subcores=NS),
        compiler_params=pltpu.CompilerParams(needs_layout_passes=False),
        scratch_types=[
            pltpu.VMEM((3 * N,), jnp.float32),
            pltpu.VMEM((WIN,), jnp.int32),
            pltpu.VMEM((AW,), jnp.int32),
            pltpu.VMEM((AW,), jnp.int32),
            pltpu.VMEM((ROWS, AW), jnp.float32),
            pltpu.SemaphoreType.DMA,
        ],
    )


def _energy_body(xj_ref, par_ref, out_ref):
    i = pl.program_id(0)
    xj = xj_ref[...]
    A = par_ref[0]
    B = par_ref[1]
    p = par_ref[2]
    q = par_ref[3]
    sigma = par_ref[4]
    gamma = par_ref[5]
    cutoff = par_ref[6]
    lam = par_ref[7]
    cb0 = par_ref[8]
    xjx = xj[0:K]
    xjy = xj[K:2 * K]
    xjz = xj[2 * K:3 * K]
    dx = xjx - xj[3 * K:3 * K + 1]
    dy = xjy - xj[3 * K + 1:3 * K + 2]
    dz = xjz - xj[3 * K + 2:3 * K + 3]
    nnf = xj[3 * K + 3:3 * K + 4]
    rij = jnp.sqrt(dx * dx + dy * dy + dz * dz)
    srow = lax.broadcasted_iota(jnp.int32, (K, BN), 0).astype(jnp.float32)
    mask = srow < nnf
    valid2 = mask & (rij < cutoff)
    r2 = jnp.where(valid2, rij, 1.0)
    sig_r = sigma / r2
    lsr = jnp.log(sig_r)
    e2 = A * (B * jnp.exp(p * lsr) - jnp.exp(q * lsr)) * jnp.exp(sigma / (r2 - cutoff))
    acc2 = jnp.where(valid2, e2, 0.0)
    E = jnp.exp(gamma / (rij - cutoff))
    Em = jnp.where(mask, E, 0.0)
    rr = rij * rij
    rrs = jnp.maximum(rr, 1e-30)
    G = Em / (2.0 * rrs)
    GL = lam * G
    tri = lax.broadcasted_iota(jnp.int32, (K, 1), 0)
    H = K // 2
    xjx_h, xjy_h, xjz_h = xjx[H:K], xjy[H:K], xjz[H:K]
    rij_h, rr_h, G_h = rij[H:K], rr[H:K], G[H:K]
    tri_h = tri[H:K]
    acc3 = jnp.zeros((K, BN), jnp.float32)
    acch = jnp.zeros((H, BN), jnp.float32)
    for s in range(K - 1):
        if s < H:
            xs, ys, zs = xjx, xjy, xjz
            rrv, rv, Gv, tv = rr, rij, G, tri
        else:
            xs, ys, zs = xjx_h, xjy_h, xjz_h
            rrv, rv, Gv, tv = rr_h, rij_h, G_h, tri_h
        ex = xs - xjx[s:s + 1]
        ey = ys - xjy[s:s + 1]
        ez = zs - xjz[s:s + 1]
        rjk2 = ex * ex + ey * ey + ez * ez
        num = rr[s:s + 1] + rrv - rjk2
        den = (2.0 * rij[s:s + 1]) * rv
        u = num - cb0 * den
        e3 = (GL[s:s + 1] * Gv) * (u * u)
        sel = jnp.where(tv > s, e3, 0.0)
        if s < H:
            acc3 = acc3 + sel
        else:
            acch = acch + sel
    acc = 0.5 * jnp.sum(acc2) + jnp.sum(acc3) + jnp.sum(acch)

    @pl.when(i == 0)
    def _():
        out_ref[0] = 0.0

    out_ref[0] += acc


def _energy_call(xj, params):
    return pl.pallas_call(
        _energy_body,
        grid=(NH // BN,),
        in_specs=[
            pl.BlockSpec((ROWS, BN), lambda i: (0, i)),
            pl.BlockSpec(memory_space=pltpu.SMEM),
        ],
        out_specs=pl.BlockSpec(memory_space=pltpu.SMEM),
        out_shape=jax.ShapeDtypeStruct((1,), jnp.float32),
    )(xj, params)


def kernel(particle_contributing, coords, num_neighbors, neighbor_list,
           A, B, p, q, sigma, gamma, cutoff, lam, cos_beta0):
    consumed = jnp.where(particle_contributing == 1, num_neighbors, 0).astype(jnp.int32)
    begin = jnp.concatenate(
        [jnp.zeros((1,), jnp.int32), jnp.cumsum(consumed)[:-1].astype(jnp.int32)])
    tab = coords.reshape(3 * N)
    nl = neighbor_list.reshape(TOTAL)
    xj = _sc_gather(0)(tab, nl, begin, consumed)
    params = jnp.stack([A, B, p, q, sigma, gamma, cutoff, lam, cos_beta0])
    out = _energy_call(xj, params)
    return out[0]


# speedup vs baseline: 1.0833x; 1.0833x over previous
import functools

import jax
import jax.numpy as jnp
from jax import lax
from jax.experimental import pallas as pl
from jax.experimental.pallas import tpu as pltpu
from jax.experimental.pallas import tpu_sc as plsc

N = 16384
K = 16
TOTAL = N * K
NC = 2
NS = 16
NW = NC * NS
NH = N
AW = NH // NW
GROUPS = AW // 16
WIN = 8192
ROWS = 52
BN = 1024


def _sc_gather_body(tab_hbm, nl_hbm, beg_hbm, con_hbm, out_hbm,
                    tab, nlw, beg, con, ob, tsem, half):
    wid = lax.axis_index("s") * NC + lax.axis_index("c")
    a0l = wid * AW
    a0 = a0l + half * NH
    tab_cp = pltpu.async_copy(tab_hbm, tab, tsem)
    pltpu.sync_copy(beg_hbm.at[pl.ds(a0, AW)], beg)
    pltpu.sync_copy(con_hbm.at[pl.ds(a0, AW)], con)
    b0 = beg[pl.ds(0, 16)][0]
    off = jnp.minimum((b0 // 8) * 8, TOTAL - WIN)
    off = pl.multiple_of(off, 8)
    pltpu.sync_copy(nl_hbm.at[pl.ds(off, WIN)], nlw)
    tab_cp.wait()
    lane = lax.iota(jnp.int32, 16)

    @plsc.parallel_loop(0, GROUPS, unroll=8)
    def group(g):
        base = g * 16
        aidx = (a0 + base + lane) * 3
        ob[3 * K + 0, pl.ds(base, 16)] = plsc.load_gather(tab, [aidx])
        ob[3 * K + 1, pl.ds(base, 16)] = plsc.load_gather(tab, [aidx + 1])
        ob[3 * K + 2, pl.ds(base, 16)] = plsc.load_gather(tab, [aidx + 2])
        ob[3 * K + 3, pl.ds(base, 16)] = con[pl.ds(base, 16)].astype(jnp.float32)
        bvec = jnp.clip(beg[pl.ds(base, 16)] - off, 0, WIN - K)
        for s in range(K):
            nlv = plsc.load_gather(nlw, [bvec + s]) * 3
            ob[s, pl.ds(base, 16)] = plsc.load_gather(tab, [nlv])
            ob[K + s, pl.ds(base, 16)] = plsc.load_gather(tab, [nlv + 1])
            ob[2 * K + s, pl.ds(base, 16)] = plsc.load_gather(tab, [nlv + 2])

    pltpu.sync_copy(ob, out_hbm.at[:, pl.ds(a0l, AW)])


@functools.lru_cache(maxsize=2)
def _sc_gather(half):
    return pl.kernel(
        functools.partial(_sc_gather_body, half=half),
        out_type=jax.ShapeDtypeStruct((ROWS, NH), jnp.float32),
        mesh=plsc.VectorSubcoreMesh(core_axis_name="c", subcore_axis_name="s",
                                    num_cores=NC, num_subcores=NS),
        compiler_params=pltpu.CompilerParams(needs_layout_passes=False),
        scratch_types=[
            pltpu.VMEM((3 * N,), jnp.float32),
            pltpu.VMEM((WIN,), jnp.int32),
            pltpu.VMEM((AW,), jnp.int32),
            pltpu.VMEM((AW,), jnp.int32),
            pltpu.VMEM((ROWS, AW), jnp.float32),
            pltpu.SemaphoreType.DMA,
        ],
    )


def _energy_body(xj_ref, par_ref, out_ref):
    i = pl.program_id(0)
    xj = xj_ref[...]
    A = par_ref[0]
    B = par_ref[1]
    p = par_ref[2]
    q = par_ref[3]
    sigma = par_ref[4]
    gamma = par_ref[5]
    cutoff = par_ref[6]
    lam = par_ref[7]
    cb0 = par_ref[8]
    xjx = xj[0:K]
    xjy = xj[K:2 * K]
    xjz = xj[2 * K:3 * K]
    dx = xjx - xj[3 * K:3 * K + 1]
    dy = xjy - xj[3 * K + 1:3 * K + 2]
    dz = xjz - xj[3 * K + 2:3 * K + 3]
    nnf = xj[3 * K + 3:3 * K + 4]
    rij = jnp.sqrt(dx * dx + dy * dy + dz * dz)
    srow = lax.broadcasted_iota(jnp.int32, (K, BN), 0).astype(jnp.float32)
    mask = srow < nnf
    valid2 = mask & (rij < cutoff)
    r2 = jnp.where(valid2, rij, 1.0)
    sig_r = sigma / r2
    lsr = jnp.log(sig_r)
    e2 = A * (B * jnp.exp(p * lsr) - jnp.exp(q * lsr)) * jnp.exp(sigma / (r2 - cutoff))
    acc2 = jnp.where(valid2, e2, 0.0)
    E = jnp.exp(gamma / (rij - cutoff))
    Em = jnp.where(mask, E, 0.0)
    rr = rij * rij
    rrs = jnp.maximum(rr, 1e-30)
    G = Em / (2.0 * rrs)
    GL = lam * G
    tri = lax.broadcasted_iota(jnp.int32, (K, 1), 0)
    H = K // 2
    xjx_h, xjy_h, xjz_h = xjx[H:K], xjy[H:K], xjz[H:K]
    rij_h, rr_h, G_h = rij[H:K], rr[H:K], G[H:K]
    tri_h = tri[H:K]
    acc3 = jnp.zeros((K, BN), jnp.float32)
    acch = jnp.zeros((H, BN), jnp.float32)
    for s in range(K - 1):
        if s < H:
            xs, ys, zs = xjx, xjy, xjz
            rrv, rv, Gv, tv = rr, rij, G, tri
        else:
            xs, ys, zs = xjx_h, xjy_h, xjz_h
            rrv, rv, Gv, tv = rr_h, rij_h, G_h, tri_h
        ex = xs - xjx[s:s + 1]
        ey = ys - xjy[s:s + 1]
        ez = zs - xjz[s:s + 1]
        rjk2 = ex * ex + ey * ey + ez * ez
        num = rr[s:s + 1] + rrv - rjk2
        den = (2.0 * rij[s:s + 1]) * rv
        u = num - cb0 * den
        e3 = (GL[s:s + 1] * Gv) * (u * u)
        sel = jnp.where(tv > s, e3, 0.0)
        if s < H:
            acc3 = acc3 + sel
        else:
            acch = acch + sel
    acc = 0.5 * jnp.sum(acc2) + jnp.sum(acc3) + jnp.sum(acch)

    @pl.when(i == 0)
    def _():
        out_ref[0] = 0.0

    out_ref[0] += acc


def _energy_call(xj, params):
    return pl.pallas_call(
        _energy_body,
        grid=(NH // BN,),
        in_specs=[
            pl.BlockSpec((ROWS, BN), lambda i: (0, i)),
            pl.BlockSpec(memory_space=pltpu.SMEM),
        ],
        out_specs=pl.BlockSpec(memory_space=pltpu.SMEM),
        out_shape=jax.ShapeDtypeStruct((1,), jnp.float32),
    )(xj, params)


def kernel(particle_contributing, coords, num_neighbors, neighbor_list,
           A, B, p, q, sigma, gamma, cutoff, lam, cos_beta0):
    consumed = jnp.where(particle_contributing == 1, num_neighbors, 0).astype(jnp.int32)
    begin = jnp.concatenate(
        [jnp.zeros((1,), jnp.int32), jnp.cumsum(consumed)[:-1].astype(jnp.int32)])
    tab = coords.reshape(3 * N)
    nl = neighbor_list.reshape(TOTAL)
    xj = _sc_gather(0)(tab, nl, begin, consumed)
    params = jnp.stack([A, B, p, q, sigma, gamma, cutoff, lam, cos_beta0])
    out = _energy_call(xj, params)
    return out[0]


# speedup vs baseline: 1.0999x; 1.0153x over previous
import functools

import jax
import jax.numpy as jnp
from jax import lax
from jax.experimental import pallas as pl
from jax.experimental.pallas import tpu as pltpu
from jax.experimental.pallas import tpu_sc as plsc

N = 16384
K = 16
TOTAL = N * K
NC = 2
NS = 16
NW = NC * NS
NH = N
AW = NH // NW
GROUPS = AW // 16
WIN = 8192
ROWS = 52
BN = 2048


def _sc_gather_body(tab_hbm, nl_hbm, beg_hbm, con_hbm, out_hbm,
                    tab, nlw, beg, con, ob, tsem, half):
    wid = lax.axis_index("s") * NC + lax.axis_index("c")
    a0l = wid * AW
    a0 = a0l + half * NH
    tab_cp = pltpu.async_copy(tab_hbm, tab, tsem)
    pltpu.sync_copy(beg_hbm.at[pl.ds(a0, AW)], beg)
    pltpu.sync_copy(con_hbm.at[pl.ds(a0, AW)], con)
    b0 = beg[pl.ds(0, 16)][0]
    off = jnp.minimum((b0 // 8) * 8, TOTAL - WIN)
    off = pl.multiple_of(off, 8)
    pltpu.sync_copy(nl_hbm.at[pl.ds(off, WIN)], nlw)
    tab_cp.wait()
    lane = lax.iota(jnp.int32, 16)

    @plsc.parallel_loop(0, GROUPS, unroll=8)
    def group(g):
        base = g * 16
        aidx = (a0 + base + lane) * 3
        ob[3 * K + 0, pl.ds(base, 16)] = plsc.load_gather(tab, [aidx])
        ob[3 * K + 1, pl.ds(base, 16)] = plsc.load_gather(tab, [aidx + 1])
        ob[3 * K + 2, pl.ds(base, 16)] = plsc.load_gather(tab, [aidx + 2])
        ob[3 * K + 3, pl.ds(base, 16)] = con[pl.ds(base, 16)].astype(jnp.float32)
        bvec = jnp.clip(beg[pl.ds(base, 16)] - off, 0, WIN - K)
        for s in range(K):
            nlv = plsc.load_gather(nlw, [bvec + s]) * 3
            ob[s, pl.ds(base, 16)] = plsc.load_gather(tab, [nlv])
            ob[K + s, pl.ds(base, 16)] = plsc.load_gather(tab, [nlv + 1])
            ob[2 * K + s, pl.ds(base, 16)] = plsc.load_gather(tab, [nlv + 2])

    pltpu.sync_copy(ob, out_hbm.at[:, pl.ds(a0l, AW)])


@functools.lru_cache(maxsize=2)
def _sc_gather(half):
    return pl.kernel(
        functools.partial(_sc_gather_body, half=half),
        out_type=jax.ShapeDtypeStruct((ROWS, NH), jnp.float32),
        mesh=plsc.VectorSubcoreMesh(core_axis_name="c", subcore_axis_name="s",
                                    num_cores=NC, num_subcores=NS),
        compiler_params=pltpu.CompilerParams(needs_layout_passes=False),
        scratch_types=[
            pltpu.VMEM((3 * N,), jnp.float32),
            pltpu.VMEM((WIN,), jnp.int32),
            pltpu.VMEM((AW,), jnp.int32),
            pltpu.VMEM((AW,), jnp.int32),
            pltpu.VMEM((ROWS, AW), jnp.float32),
            pltpu.SemaphoreType.DMA,
        ],
    )


def _energy_body(xj_ref, par_ref, out_ref):
    i = pl.program_id(0)
    xj = xj_ref[...]
    A = par_ref[0]
    B = par_ref[1]
    p = par_ref[2]
    q = par_ref[3]
    sigma = par_ref[4]
    gamma = par_ref[5]
    cutoff = par_ref[6]
    lam = par_ref[7]
    cb0 = par_ref[8]
    xjx = xj[0:K]
    xjy = xj[K:2 * K]
    xjz = xj[2 * K:3 * K]
    dx = xjx - xj[3 * K:3 * K + 1]
    dy = xjy - xj[3 * K + 1:3 * K + 2]
    dz = xjz - xj[3 * K + 2:3 * K + 3]
    nnf = xj[3 * K + 3:3 * K + 4]
    rij = jnp.sqrt(dx * dx + dy * dy + dz * dz)
    srow = lax.broadcasted_iota(jnp.int32, (K, BN), 0).astype(jnp.float32)
    mask = srow < nnf
    valid2 = mask & (rij < cutoff)
    r2 = jnp.where(valid2, rij, 1.0)
    sig_r = sigma / r2
    lsr = jnp.log(sig_r)
    e2 = A * (B * jnp.exp(p * lsr) - jnp.exp(q * lsr)) * jnp.exp(sigma / (r2 - cutoff))
    acc2 = jnp.where(valid2, e2, 0.0)
    E = jnp.exp(gamma / (rij - cutoff))
    Em = jnp.where(mask, E, 0.0)
    rr = rij * rij
    rrs = jnp.maximum(rr, 1e-30)
    G = Em / (2.0 * rrs)
    GL = lam * G
    tri = lax.broadcasted_iota(jnp.int32, (K, 1), 0)
    H = K // 2
    xjx_h, xjy_h, xjz_h = xjx[H:K], xjy[H:K], xjz[H:K]
    rij_h, rr_h, G_h = rij[H:K], rr[H:K], G[H:K]
    tri_h = tri[H:K]
    acc3 = jnp.zeros((K, BN), jnp.float32)
    acch = jnp.zeros((H, BN), jnp.float32)
    for s in range(K - 1):
        if s < H:
            xs, ys, zs = xjx, xjy, xjz
            rrv, rv, Gv, tv = rr, rij, G, tri
        else:
            xs, ys, zs = xjx_h, xjy_h, xjz_h
            rrv, rv, Gv, tv = rr_h, rij_h, G_h, tri_h
        ex = xs - xjx[s:s + 1]
        ey = ys - xjy[s:s + 1]
        ez = zs - xjz[s:s + 1]
        rjk2 = ex * ex + ey * ey + ez * ez
        num = rr[s:s + 1] + rrv - rjk2
        den = (2.0 * rij[s:s + 1]) * rv
        u = num - cb0 * den
        e3 = (GL[s:s + 1] * Gv) * (u * u)
        sel = jnp.where(tv > s, e3, 0.0)
        if s < H:
            acc3 = acc3 + sel
        else:
            acch = acch + sel
    acc = 0.5 * jnp.sum(acc2) + jnp.sum(acc3) + jnp.sum(acch)

    @pl.when(i == 0)
    def _():
        out_ref[0] = 0.0

    out_ref[0] += acc


def _energy_call(xj, params):
    return pl.pallas_call(
        _energy_body,
        grid=(NH // BN,),
        in_specs=[
            pl.BlockSpec((ROWS, BN), lambda i: (0, i)),
            pl.BlockSpec(memory_space=pltpu.SMEM),
        ],
        out_specs=pl.BlockSpec(memory_space=pltpu.SMEM),
        out_shape=jax.ShapeDtypeStruct((1,), jnp.float32),
    )(xj, params)


def kernel(particle_contributing, coords, num_neighbors, neighbor_list,
           A, B, p, q, sigma, gamma, cutoff, lam, cos_beta0):
    consumed = jnp.where(particle_contributing == 1, num_neighbors, 0).astype(jnp.int32)
    begin = jnp.concatenate(
        [jnp.zeros((1,), jnp.int32), jnp.cumsum(consumed)[:-1].astype(jnp.int32)])
    tab = coords.reshape(3 * N)
    nl = neighbor_list.reshape(TOTAL)
    xj = _sc_gather(0)(tab, nl, begin, consumed)
    params = jnp.stack([A, B, p, q, sigma, gamma, cutoff, lam, cos_beta0])
    out = _energy_call(xj, params)
    return out[0]
